# baseline (speedup 1.0000x reference)
import functools

import jax
import jax.numpy as jnp
from jax import lax
from jax.experimental import pallas as pl
from jax.experimental.pallas import tpu as pltpu
from jax.experimental.pallas import tpu_sc as plsc

_NUM_ARMS = 1000000
_D = 32
_B = 4096
_S = 50

_NW = 32
_CB = 8
_T = _CB * _S
_G = 4
_TPG = _T // _G
_CHUNKS = _B // _CB
_CPW = _CHUNKS // _NW
_OUT_F = _T * 3 * _D


def _lane_bcast(vec, j):
    idx = jnp.full((16, 1), j, dtype=jnp.int32)
    dnums = lax.GatherDimensionNumbers(
        offset_dims=(), collapsed_slice_dims=(0,), start_index_map=(0,))
    return lax.gather(vec, idx, dnums, slice_sizes=(1,),
                      mode=lax.GatherScatterMode.PROMISE_IN_BOUNDS)


def _sc_body(state_hbm, action_hbm, reward_hbm, wo_hbm, bo_hbm, emb_hbm,
             wr_hbm, br_hbm, out_hbm,
             idx_v, gath_v, sv_v, rv_v, par_v, out_v, sem):
    wid = lax.axis_index("s") * 2 + lax.axis_index("c")

    pltpu.sync_copy(wo_hbm, par_v.at[0])
    pltpu.sync_copy(bo_hbm, par_v.at[1])
    pltpu.sync_copy(wr_hbm, par_v.at[2])
    pltpu.sync_copy(br_hbm, par_v.at[3])
    wo_lo = par_v[0, pl.ds(0, 16)]
    wo_hi = par_v[0, pl.ds(16, 16)]
    bo_lo = par_v[1, pl.ds(0, 16)]
    bo_hi = par_v[1, pl.ds(16, 16)]
    wr_lo = par_v[2, pl.ds(0, 16)]
    wr_hi = par_v[2, pl.ds(16, 16)]
    br_lo = par_v[3, pl.ds(0, 16)]
    br_hi = par_v[3, pl.ds(16, 16)]

    def chunk_body(i, carry):
        c = wid * _CPW + i
        pltpu.sync_copy(action_hbm.at[pl.ds(c * _G, _G)], idx_v)
        pltpu.sync_copy(state_hbm.at[pl.ds(c * _T, _T)], sv_v)
        pltpu.sync_copy(reward_hbm.at[pl.ds(c * _T, _T)], rv_v)
        descs = [
            pltpu.async_copy(emb_hbm.at[idx_v.at[g]],
                             gath_v.at[pl.ds(g * _TPG, _TPG)], sem)
            for g in range(_G)
        ]
        for d in descs:
            d.wait()

        def group_body(tg, carry2):
            sv = sv_v[pl.ds(tg * 16, 16)]
            rv = rv_v[pl.ds(tg * 16, 16)]
            for j in range(16):
                t = tg * 16 + j
                base = t * 96
                sb = _lane_bcast(sv, j)
                rb = _lane_bcast(rv, j)
                out_v[pl.ds(base, 16)] = sb * wo_lo + bo_lo
                out_v[pl.ds(base + 16, 16)] = sb * wo_hi + bo_hi
                out_v[pl.ds(base + 32, 16)] = gath_v[t, pl.ds(0, 16)]
                out_v[pl.ds(base + 48, 16)] = gath_v[t, pl.ds(16, 16)]
                out_v[pl.ds(base + 64, 16)] = rb * wr_lo + br_lo
                out_v[pl.ds(base + 80, 16)] = rb * wr_hi + br_hi
            return carry2

        lax.fori_loop(0, _T // 16, group_body, 0)
        pltpu.sync_copy(out_v, out_hbm.at[pl.ds(c * _OUT_F, _OUT_F)])
        return carry

    lax.fori_loop(0, _CPW, chunk_body, 0)


@jax.jit
def _encode(state_f, action_r, reward_f, wo, bo, emb_table, wr, br):
    mesh = plsc.VectorSubcoreMesh(core_axis_name="c", subcore_axis_name="s")
    call = pl.kernel(
        _sc_body,
        out_type=jax.ShapeDtypeStruct((_B * 3 * _S * _D,), jnp.float32),
        mesh=mesh,
        compiler_params=pltpu.CompilerParams(use_tc_tiling_on_sc=False),
        scratch_types=[
            pltpu.VMEM((_G, _TPG), jnp.int32),
            pltpu.VMEM((_T, _D), jnp.float32),
            pltpu.VMEM((_T,), jnp.float32),
            pltpu.VMEM((_T,), jnp.float32),
            pltpu.VMEM((4, _D), jnp.float32),
            pltpu.VMEM((_OUT_F,), jnp.float32),
            pltpu.SemaphoreType.DMA,
        ],
    )
    return call(state_f, action_r, reward_f, wo, bo, emb_table, wr, br)


def kernel(state, action, reward, W_obs, b_obs, emb_table, W_rew, b_rew):
    state_f = state.reshape(_B * _S)
    reward_f = reward.reshape(_B * _S)
    action_r = action.reshape(_B * _S // _TPG, _TPG).astype(jnp.int32)
    out = _encode(state_f, action_r, reward_f,
                  W_obs.reshape(_D), b_obs, emb_table, W_rew.reshape(_D), b_rew)
    return out.reshape(_B, 3 * _S, _D)


# speedup vs baseline: 1.0866x; 1.0866x over previous
import jax
import jax.numpy as jnp
from jax import lax
from jax.experimental import pallas as pl
from jax.experimental.pallas import tpu as pltpu
from jax.experimental.pallas import tpu_sc as plsc

_NUM_ARMS = 1000000
_D = 32
_B = 4096
_S = 50

_NW = 32
_CB = 8
_T = _CB * _S
_G = 4
_TPG = _T // _G
_CHUNKS = _B // _CB
_CPW = _CHUNKS // _NW
_PAIRS = _CPW // 2


def _lane_bcast(vec, j):
    idx = jnp.full((16, 1), j, dtype=jnp.int32)
    dnums = lax.GatherDimensionNumbers(
        offset_dims=(), collapsed_slice_dims=(0,), start_index_map=(0,))
    return lax.gather(vec, idx, dnums, slice_sizes=(1,),
                      mode=lax.GatherScatterMode.PROMISE_IN_BOUNDS)


def _sc_body(state_hbm, action_hbm, reward_hbm, wo_hbm, bo_hbm, emb_hbm,
             wr_hbm, br_hbm, out_hbm,
             idx0, idx1, sv0, sv1, rv0, rv1, g0, g1, ob0, ob1, rw0, rw1,
             par_v, sem_in0, sem_in1, sem_g0, sem_g1, sem_o0, sem_o1):
    idx = (idx0, idx1)
    sv = (sv0, sv1)
    rv = (rv0, rv1)
    gb = (g0, g1)
    ob = (ob0, ob1)
    rw = (rw0, rw1)
    sem_in = (sem_in0, sem_in1)
    sem_g = (sem_g0, sem_g1)
    sem_o = (sem_o0, sem_o1)

    wid = lax.axis_index("s") * 2 + lax.axis_index("c")
    base = wid * _CPW

    pltpu.sync_copy(wo_hbm, par_v.at[0])
    pltpu.sync_copy(bo_hbm, par_v.at[1])
    pltpu.sync_copy(wr_hbm, par_v.at[2])
    pltpu.sync_copy(br_hbm, par_v.at[3])
    wo_lo = par_v[0, pl.ds(0, 16)]
    wo_hi = par_v[0, pl.ds(16, 16)]
    bo_lo = par_v[1, pl.ds(0, 16)]
    bo_hi = par_v[1, pl.ds(16, 16)]
    wr_lo = par_v[2, pl.ds(0, 16)]
    wr_hi = par_v[2, pl.ds(16, 16)]
    br_lo = par_v[3, pl.ds(0, 16)]
    br_hi = par_v[3, pl.ds(16, 16)]

    def fire_in(s, c):
        pltpu.async_copy(action_hbm.at[pl.ds(c * _G, _G)], idx[s], sem_in[s])
        pltpu.async_copy(state_hbm.at[pl.ds(c * _T, _T)], sv[s], sem_in[s])
        pltpu.async_copy(reward_hbm.at[pl.ds(c * _T, _T)], rv[s], sem_in[s])

    def wait_in(s):
        pltpu.make_async_copy(action_hbm.at[pl.ds(0, _G)], idx[s],
                              sem_in[s]).wait()
        pltpu.make_async_copy(state_hbm.at[pl.ds(0, _T)], sv[s],
                              sem_in[s]).wait()
        pltpu.make_async_copy(reward_hbm.at[pl.ds(0, _T)], rv[s],
                              sem_in[s]).wait()

    def fire_out(s, c):
        pltpu.async_copy(ob[s], out_hbm.at[pl.ds(c * _T, _T), 0], sem_o[s])
        pltpu.async_copy(gb[s], out_hbm.at[pl.ds(c * _T, _T), 1], sem_o[s])
        pltpu.async_copy(rw[s], out_hbm.at[pl.ds(c * _T, _T), 2], sem_o[s])

    def wait_out(s):
        pltpu.make_async_copy(ob[s], out_hbm.at[pl.ds(0, _T), 0],
                              sem_o[s]).wait()
        pltpu.make_async_copy(gb[s], out_hbm.at[pl.ds(0, _T), 1],
                              sem_o[s]).wait()
        pltpu.make_async_copy(rw[s], out_hbm.at[pl.ds(0, _T), 2],
                              sem_o[s]).wait()

    def compute(s):
        ob_s = ob[s]
        rw_s = rw[s]
        sv_s = sv[s]
        rv_s = rv[s]

        def group_body(tg, carry):
            sv16 = sv_s[pl.ds(tg * 16, 16)]
            rv16 = rv_s[pl.ds(tg * 16, 16)]
            for j in range(16):
                t = tg * 16 + j
                sb = _lane_bcast(sv16, j)
                rb = _lane_bcast(rv16, j)
                ob_s[t, pl.ds(0, 16)] = sb * wo_lo + bo_lo
                ob_s[t, pl.ds(16, 16)] = sb * wo_hi + bo_hi
                rw_s[t, pl.ds(0, 16)] = rb * wr_lo + br_lo
                rw_s[t, pl.ds(16, 16)] = rb * wr_hi + br_hi
            return carry

        lax.fori_loop(0, _T // 16, group_body, 0)

    fire_in(0, base)
    fire_in(1, base + 1)

    def pair_body(ip, carry):
        for s in (0, 1):
            c = base + 2 * ip + s

            @pl.when(ip >= 1)
            def _():
                wait_out(s)

            wait_in(s)
            descs = [
                pltpu.async_copy(emb_hbm.at[idx[s].at[g]],
                                 gb[s].at[pl.ds(g * _TPG, _TPG)], sem_g[s])
                for g in range(_G)
            ]
            compute(s)
            for d in descs:
                d.wait()

            @pl.when(ip < _PAIRS - 1)
            def _():
                fire_in(s, c + 2)

            fire_out(s, c)
        return carry

    lax.fori_loop(0, _PAIRS, pair_body, 0)
    wait_out(0)
    wait_out(1)


@jax.jit
def _encode(state_f, action_r, reward_f, wo, bo, emb_table, wr, br):
    mesh = plsc.VectorSubcoreMesh(core_axis_name="c", subcore_axis_name="s")
    call = pl.kernel(
        _sc_body,
        out_type=jax.ShapeDtypeStruct((_B * _S, 3, _D), jnp.float32),
        mesh=mesh,
        compiler_params=pltpu.CompilerParams(use_tc_tiling_on_sc=False),
        scratch_types=[
            pltpu.VMEM((_G, _TPG), jnp.int32),
            pltpu.VMEM((_G, _TPG), jnp.int32),
            pltpu.VMEM((_T,), jnp.float32),
            pltpu.VMEM((_T,), jnp.float32),
            pltpu.VMEM((_T,), jnp.float32),
            pltpu.VMEM((_T,), jnp.float32),
            pltpu.VMEM((_T, _D), jnp.float32),
            pltpu.VMEM((_T, _D), jnp.float32),
            pltpu.VMEM((_T, _D), jnp.float32),
            pltpu.VMEM((_T, _D), jnp.float32),
            pltpu.VMEM((_T, _D), jnp.float32),
            pltpu.VMEM((_T, _D), jnp.float32),
            pltpu.VMEM((4, _D), jnp.float32),
            pltpu.SemaphoreType.DMA,
            pltpu.SemaphoreType.DMA,
            pltpu.SemaphoreType.DMA,
            pltpu.SemaphoreType.DMA,
            pltpu.SemaphoreType.DMA,
            pltpu.SemaphoreType.DMA,
        ],
    )
    return call(state_f, action_r, reward_f, wo, bo, emb_table, wr, br)


def kernel(state, action, reward, W_obs, b_obs, emb_table, W_rew, b_rew):
    state_f = state.reshape(_B * _S)
    reward_f = reward.reshape(_B * _S)
    action_r = action.reshape(_B * _S // _TPG, _TPG).astype(jnp.int32)
    out = _encode(state_f, action_r, reward_f,
                  W_obs.reshape(_D), b_obs, emb_table, W_rew.reshape(_D), b_rew)
    return out.reshape(_B, 3 * _S, _D)
